# baseline (speedup 1.0000x reference)
import functools

import jax
import jax.numpy as jnp
from jax import lax
from jax.experimental import pallas as pl
from jax.experimental.pallas import tpu as pltpu
from jax.experimental.pallas import tpu_sc as plsc

N = 10000
E = 320000
H = 128

NC = 2
NS = 16
LANES = 16

NP = 10240
TRASH = 10200

C = 128
CA = 64
CHUNKS_PER_TILE = 160
IB = 40
NROWS_A = NC * NS * CHUNKS_PER_TILE
E_PAD = NROWS_A * CA
NROWS = E_PAD // C
ROWS_PER_TILE_DEG = NROWS // NS

_mesh = plsc.VectorSubcoreMesh(core_axis_name="c", subcore_axis_name="s")


def _zero_vmem_rows(ref, nrows):
    def body(i, _):
        for k in range(H // LANES):
            ref[i, pl.ds(k * LANES, LANES)] = jnp.zeros((LANES,), jnp.float32)
        return _
    lax.fori_loop(0, nrows, body, None)


_DEG_K = 16

def _deg_body(edges_hbm, out_hbm, hist_sh, ones_v, zeros_v, idx_a, cp_v, sem):
    cid = lax.axis_index("c")
    sid = lax.axis_index("s")

    for k in range(C // LANES):
        ones_v[pl.ds(k * LANES, LANES)] = jnp.ones((LANES,), jnp.float32)
        zeros_v[pl.ds(k * LANES, LANES)] = jnp.zeros((LANES,), jnp.float32)

    slab = NP // NS
    base = sid * slab
    for k in range(slab // C):
        pltpu.sync_copy(zeros_v, hist_sh.at[pl.ds(base + k * C, C)])
    plsc.subcore_barrier()

    rbase = sid * ROWS_PER_TILE_DEG
    pltpu.sync_copy(edges_hbm.at[cid, pl.ds(rbase, ROWS_PER_TILE_DEG)], idx_a)
    def blk(b, carry):
        rb = b * _DEG_K
        for k in range(_DEG_K):
            pltpu.async_copy(ones_v, hist_sh.at[idx_a.at[rb + k]], sem, add=True)
        for k in range(_DEG_K):
            pltpu.make_async_copy(ones_v, hist_sh.at[idx_a.at[rb + k]], sem).wait()
        return carry
    lax.fori_loop(0, ROWS_PER_TILE_DEG // _DEG_K, blk, None)
    plsc.subcore_barrier()

    pltpu.sync_copy(hist_sh.at[pl.ds(base, slab)], cp_v)
    pltpu.sync_copy(cp_v, out_hbm.at[cid, pl.ds(base, slab)])


_deg_call = pl.kernel(
    _deg_body,
    out_type=jax.ShapeDtypeStruct((2, NP), jnp.float32),
    mesh=_mesh,
    scratch_types=[
        pltpu.VMEM_SHARED((NP,), jnp.float32),
        pltpu.VMEM((C,), jnp.float32),
        pltpu.VMEM((C,), jnp.float32),
        pltpu.VMEM((ROWS_PER_TILE_DEG, C), jnp.int32),
        pltpu.VMEM((NP // NS,), jnp.float32),
        pltpu.SemaphoreType.DMA,
    ],
)


def _agg_body(h_hbm, edges_hbm, out_hbm, acc_sh, rows0, rows1, rows2, rows3,
              sidx_a, didx_a, sem0, sem1, sem2, sem3):
    cid = lax.axis_index("c")
    sid = lax.axis_index("s")
    wid = cid * NS + sid

    _zero_vmem_rows(rows0, CA)
    slab = NP // NS
    base = sid * slab
    for k in range(slab // CA):
        pltpu.sync_copy(rows0, acc_sh.at[pl.ds(base + k * CA, CA)])
    plsc.subcore_barrier()

    rows = [rows0, rows1, rows2, rows3]
    sems = [sem0, sem1, sem2, sem3]
    wbase = wid * CHUNKS_PER_TILE
    def block(b, carry):
        bbase = wbase + b * IB
        pltpu.sync_copy(edges_hbm.at[0, pl.ds(bbase, IB)], sidx_a)
        pltpu.sync_copy(edges_hbm.at[1, pl.ds(bbase, IB)], didx_a)
        for k in range(3):
            pltpu.async_copy(h_hbm.at[sidx_a.at[k]], rows[k], sems[k])
        def quad(i, c2):
            j0 = 4 * i
            for k in range(4):
                j = j0 + k
                pltpu.make_async_copy(h_hbm.at[sidx_a.at[j]], rows[k], sems[k]).wait()
                pltpu.sync_copy(rows[k], acc_sh.at[didx_a.at[j]], add=True)
                @pl.when(j + 3 < IB)
                def _():
                    pltpu.async_copy(h_hbm.at[sidx_a.at[j + 3]],
                                     rows[(k + 3) % 4], sems[(k + 3) % 4])
            return c2
        lax.fori_loop(0, IB // 4, quad, None)
        return carry
    lax.fori_loop(0, CHUNKS_PER_TILE // IB, block, None)
    plsc.subcore_barrier()

    for k in range(slab // CA):
        pltpu.sync_copy(acc_sh.at[pl.ds(base + k * CA, CA)], rows0)
        pltpu.sync_copy(rows0, out_hbm.at[cid, pl.ds(base + k * CA, CA)])


_agg_call = pl.kernel(
    _agg_body,
    out_type=jax.ShapeDtypeStruct((2, NP, H), jnp.float32),
    mesh=_mesh,
    scratch_types=[
        pltpu.VMEM_SHARED((NP, H), jnp.float32),
        pltpu.VMEM((CA, H), jnp.float32),
        pltpu.VMEM((CA, H), jnp.float32),
        pltpu.VMEM((CA, H), jnp.float32),
        pltpu.VMEM((CA, H), jnp.float32),
        pltpu.VMEM((IB, CA), jnp.int32),
        pltpu.VMEM((IB, CA), jnp.int32),
        pltpu.SemaphoreType.DMA,
        pltpu.SemaphoreType.DMA,
        pltpu.SemaphoreType.DMA,
        pltpu.SemaphoreType.DMA,
    ],
)


_BM = 1280
_G = NP // _BM

def _mm_body(x_ref, w_ref, o_ref):
    o_ref[...] = jnp.dot(x_ref[...], w_ref[...], preferred_element_type=jnp.float32,
                         precision=lax.Precision.HIGHEST)


def _scale_body(h_ref, deg_ref, o_ref):
    ns = lax.rsqrt(jnp.maximum(deg_ref[0, 0, :], 1.0))
    o_ref[...] = h_ref[...] * ns[:, None]


def _layer2_body(p_ref, degd_ref, degs_ref, b_ref, w_ref, o_ref):
    s = p_ref[0] + p_ref[1]
    nd = lax.rsqrt(jnp.maximum(degd_ref[0, 0, :], 1.0))
    h1 = jnp.maximum(s * nd[:, None] + b_ref[...], 0.0)
    ns = lax.rsqrt(jnp.maximum(degs_ref[0, 0, :], 1.0))
    h2 = jnp.dot(h1, w_ref[...], preferred_element_type=jnp.float32,
                 precision=lax.Precision.HIGHEST)
    o_ref[...] = h2 * ns[:, None]


def _z_body(p_ref, degd_ref, b_ref, o_ref):
    s = p_ref[0] + p_ref[1]
    nd = lax.rsqrt(jnp.maximum(degd_ref[0, 0, :], 1.0))
    o_ref[...] = s * nd[:, None] + b_ref[...]


_DBM = 2000
_DBN = 2560

def _gram_body(a_ref, b_ref, o_ref):
    o_ref[...] = lax.dot_general(
        a_ref[...], b_ref[...], (((1,), (1,)), ((), ())),
        preferred_element_type=jnp.float32, precision=lax.Precision.DEFAULT)


def kernel(in_feat, edge_index, W1, b1, W2, b2):
    x_pad = jnp.pad(in_feat, ((0, NP - N), (0, 0)))
    pad = N + jnp.arange(E_PAD - E, dtype=jnp.int32) % (NP - N)
    src_f = jnp.concatenate([edge_index[0], pad])
    dst_f = jnp.concatenate([edge_index[1], pad])
    edges = jnp.stack([src_f.reshape(NROWS, C), dst_f.reshape(NROWS, C)])
    edges_a = jnp.stack([src_f.reshape(NROWS_A, CA), dst_f.reshape(NROWS_A, CA)])

    deg = _deg_call(edges)
    deg_src = deg[0].reshape(_G, 1, _BM)
    deg_dst = deg[1].reshape(_G, 1, _BM)

    h_lin = pl.pallas_call(
        _mm_body,
        grid=(_G,),
        in_specs=[
            pl.BlockSpec((_BM, H), lambda i: (i, 0)),
            pl.BlockSpec((H, H), lambda i: (0, 0)),
        ],
        out_specs=pl.BlockSpec((_BM, H), lambda i: (i, 0)),
        out_shape=jax.ShapeDtypeStruct((NP, H), jnp.float32),
    )(x_pad, W1)

    h1s = pl.pallas_call(
        _scale_body,
        grid=(_G,),
        in_specs=[
            pl.BlockSpec((_BM, H), lambda i: (i, 0)),
            pl.BlockSpec((1, 1, _BM), lambda i: (i, 0, 0)),
        ],
        out_specs=pl.BlockSpec((_BM, H), lambda i: (i, 0)),
        out_shape=jax.ShapeDtypeStruct((NP, H), jnp.float32),
    )(h_lin, deg_src)

    agg1 = _agg_call(h1s, edges_a)

    h2s = pl.pallas_call(
        _layer2_body,
        grid=(_G,),
        in_specs=[
            pl.BlockSpec((2, _BM, H), lambda i: (0, i, 0)),
            pl.BlockSpec((1, 1, _BM), lambda i: (i, 0, 0)),
            pl.BlockSpec((1, 1, _BM), lambda i: (i, 0, 0)),
            pl.BlockSpec((1, H), lambda i: (0, 0)),
            pl.BlockSpec((H, H), lambda i: (0, 0)),
        ],
        out_specs=pl.BlockSpec((_BM, H), lambda i: (i, 0)),
        out_shape=jax.ShapeDtypeStruct((NP, H), jnp.float32),
    )(agg1, deg_dst, deg_src, b1.reshape(1, H), W2)

    agg2 = _agg_call(h2s, edges_a)

    z_pad = pl.pallas_call(
        _z_body,
        grid=(_G,),
        in_specs=[
            pl.BlockSpec((2, _BM, H), lambda i: (0, i, 0)),
            pl.BlockSpec((1, 1, _BM), lambda i: (i, 0, 0)),
            pl.BlockSpec((1, H), lambda i: (0, 0)),
        ],
        out_specs=pl.BlockSpec((_BM, H), lambda i: (i, 0)),
        out_shape=jax.ShapeDtypeStruct((NP, H), jnp.float32),
    )(agg2, deg_dst, b2.reshape(1, H))

    adj = pl.pallas_call(
        _gram_body,
        grid=(N // _DBM, pl.cdiv(N, _DBN)),
        in_specs=[
            pl.BlockSpec((_DBM, H), lambda i, j: (i, 0)),
            pl.BlockSpec((_DBN, H), lambda i, j: (j, 0)),
        ],
        out_specs=pl.BlockSpec((_DBM, _DBN), lambda i, j: (i, j)),
        out_shape=jax.ShapeDtypeStruct((N, N), jnp.float32),
    )(z_pad, z_pad)
    return adj


# speedup vs baseline: 1.0176x; 1.0176x over previous
import functools

import jax
import jax.numpy as jnp
from jax import lax
from jax.experimental import pallas as pl
from jax.experimental.pallas import tpu as pltpu
from jax.experimental.pallas import tpu_sc as plsc

N = 10000
E = 320000
H = 128

NC = 2
NS = 16
LANES = 16

NP = 10240
TRASH = 10200

C = 128
CHUNKS_PER_TILE = 80
IB = 40
NROWS = NC * NS * CHUNKS_PER_TILE
E_PAD = NROWS * C
ROWS_PER_TILE_DEG = NROWS // NS

_mesh = plsc.VectorSubcoreMesh(core_axis_name="c", subcore_axis_name="s")


def _zero_vmem_rows(ref, nrows):
    def body(i, _):
        for k in range(H // LANES):
            ref[i, pl.ds(k * LANES, LANES)] = jnp.zeros((LANES,), jnp.float32)
        return _
    lax.fori_loop(0, nrows, body, None)


_DEG_K = 16

def _deg_body(edges_hbm, out_hbm, hist_sh, ones_v, zeros_v, idx_a, cp_v, sem):
    cid = lax.axis_index("c")
    sid = lax.axis_index("s")

    for k in range(C // LANES):
        ones_v[pl.ds(k * LANES, LANES)] = jnp.ones((LANES,), jnp.float32)
        zeros_v[pl.ds(k * LANES, LANES)] = jnp.zeros((LANES,), jnp.float32)

    slab = NP // NS
    base = sid * slab
    for k in range(slab // C):
        pltpu.sync_copy(zeros_v, hist_sh.at[pl.ds(base + k * C, C)])
    plsc.subcore_barrier()

    rbase = sid * ROWS_PER_TILE_DEG
    pltpu.sync_copy(edges_hbm.at[cid, pl.ds(rbase, ROWS_PER_TILE_DEG)], idx_a)
    def blk(b, carry):
        rb = b * _DEG_K
        for k in range(_DEG_K):
            pltpu.async_copy(ones_v, hist_sh.at[idx_a.at[rb + k]], sem, add=True)
        for k in range(_DEG_K):
            pltpu.make_async_copy(ones_v, hist_sh.at[idx_a.at[rb + k]], sem).wait()
        return carry
    lax.fori_loop(0, ROWS_PER_TILE_DEG // _DEG_K, blk, None)
    plsc.subcore_barrier()

    pltpu.sync_copy(hist_sh.at[pl.ds(base, slab)], cp_v)
    pltpu.sync_copy(cp_v, out_hbm.at[cid, pl.ds(base, slab)])


_deg_call = pl.kernel(
    _deg_body,
    out_type=jax.ShapeDtypeStruct((2, NP), jnp.float32),
    mesh=_mesh,
    scratch_types=[
        pltpu.VMEM_SHARED((NP,), jnp.float32),
        pltpu.VMEM((C,), jnp.float32),
        pltpu.VMEM((C,), jnp.float32),
        pltpu.VMEM((ROWS_PER_TILE_DEG, C), jnp.int32),
        pltpu.VMEM((NP // NS,), jnp.float32),
        pltpu.SemaphoreType.DMA,
    ],
)


def _agg_body(h_hbm, edges_hbm, out_hbm, acc_sh, rows0, rows1,
              sidx_a, didx_a, sem0, sem1):
    cid = lax.axis_index("c")
    sid = lax.axis_index("s")
    wid = cid * NS + sid

    _zero_vmem_rows(rows0, C)
    slab = NP // NS
    base = sid * slab
    for k in range(slab // C):
        pltpu.sync_copy(rows0, acc_sh.at[pl.ds(base + k * C, C)])
    plsc.subcore_barrier()

    wbase = wid * CHUNKS_PER_TILE
    def block(b, carry):
        bbase = wbase + b * IB
        pltpu.sync_copy(edges_hbm.at[0, pl.ds(bbase, IB)], sidx_a)
        pltpu.sync_copy(edges_hbm.at[1, pl.ds(bbase, IB)], didx_a)
        pltpu.async_copy(h_hbm.at[sidx_a.at[0]], rows0, sem0)
        def pair(i, c2):
            j0 = 2 * i
            pltpu.async_copy(h_hbm.at[sidx_a.at[j0 + 1]], rows1, sem1)
            pltpu.make_async_copy(h_hbm.at[sidx_a.at[j0]], rows0, sem0).wait()
            pltpu.sync_copy(rows0, acc_sh.at[didx_a.at[j0]], add=True)
            @pl.when(j0 + 2 < IB)
            def _():
                pltpu.async_copy(h_hbm.at[sidx_a.at[j0 + 2]], rows0, sem0)
            pltpu.make_async_copy(h_hbm.at[sidx_a.at[j0 + 1]], rows1, sem1).wait()
            pltpu.sync_copy(rows1, acc_sh.at[didx_a.at[j0 + 1]], add=True)
            return c2
        lax.fori_loop(0, IB // 2, pair, None)
        return carry
    lax.fori_loop(0, CHUNKS_PER_TILE // IB, block, None)
    plsc.subcore_barrier()

    for k in range(slab // C):
        pltpu.sync_copy(acc_sh.at[pl.ds(base + k * C, C)], rows0)
        pltpu.sync_copy(rows0, out_hbm.at[cid, pl.ds(base + k * C, C)])


_agg_call = pl.kernel(
    _agg_body,
    out_type=jax.ShapeDtypeStruct((2, NP, H), jnp.float32),
    mesh=_mesh,
    scratch_types=[
        pltpu.VMEM_SHARED((NP, H), jnp.float32),
        pltpu.VMEM((C, H), jnp.float32),
        pltpu.VMEM((C, H), jnp.float32),
        pltpu.VMEM((IB, C), jnp.int32),
        pltpu.VMEM((IB, C), jnp.int32),
        pltpu.SemaphoreType.DMA,
        pltpu.SemaphoreType.DMA,
    ],
)


_BM = 1280
_G = NP // _BM

def _mm_body(x_ref, w_ref, o_ref):
    o_ref[...] = jnp.dot(x_ref[...], w_ref[...], preferred_element_type=jnp.float32,
                         precision=lax.Precision.HIGHEST)


def _scale_body(h_ref, deg_ref, o_ref):
    ns = lax.rsqrt(jnp.maximum(deg_ref[0, 0, :], 1.0))
    o_ref[...] = h_ref[...] * ns[:, None]


def _layer2_body(p_ref, degd_ref, degs_ref, b_ref, w_ref, o_ref):
    s = p_ref[0] + p_ref[1]
    nd = lax.rsqrt(jnp.maximum(degd_ref[0, 0, :], 1.0))
    h1 = jnp.maximum(s * nd[:, None] + b_ref[...], 0.0)
    ns = lax.rsqrt(jnp.maximum(degs_ref[0, 0, :], 1.0))
    h2 = jnp.dot(h1, w_ref[...], preferred_element_type=jnp.float32,
                 precision=lax.Precision.HIGHEST)
    o_ref[...] = h2 * ns[:, None]


def _z_body(p_ref, degd_ref, b_ref, o_ref):
    s = p_ref[0] + p_ref[1]
    nd = lax.rsqrt(jnp.maximum(degd_ref[0, 0, :], 1.0))
    o_ref[...] = s * nd[:, None] + b_ref[...]


_DBM = 2000
_DBN = 2560

def _gram_body(a_ref, b_ref, o_ref):
    o_ref[...] = lax.dot_general(
        a_ref[...], b_ref[...], (((1,), (1,)), ((), ())),
        preferred_element_type=jnp.float32, precision=lax.Precision.DEFAULT)


def kernel(in_feat, edge_index, W1, b1, W2, b2):
    x_pad = jnp.pad(in_feat, ((0, NP - N), (0, 0)))
    pad = N + jnp.arange(E_PAD - E, dtype=jnp.int32) % (NP - N)
    src_p = jnp.concatenate([edge_index[0], pad]).reshape(NROWS, C)
    dst_p = jnp.concatenate([edge_index[1], pad]).reshape(NROWS, C)
    edges = jnp.stack([src_p, dst_p])

    deg = _deg_call(edges)
    deg_src = deg[0].reshape(_G, 1, _BM)
    deg_dst = deg[1].reshape(_G, 1, _BM)

    h_lin = pl.pallas_call(
        _mm_body,
        grid=(_G,),
        in_specs=[
            pl.BlockSpec((_BM, H), lambda i: (i, 0)),
            pl.BlockSpec((H, H), lambda i: (0, 0)),
        ],
        out_specs=pl.BlockSpec((_BM, H), lambda i: (i, 0)),
        out_shape=jax.ShapeDtypeStruct((NP, H), jnp.float32),
    )(x_pad, W1)

    h1s = pl.pallas_call(
        _scale_body,
        grid=(_G,),
        in_specs=[
            pl.BlockSpec((_BM, H), lambda i: (i, 0)),
            pl.BlockSpec((1, 1, _BM), lambda i: (i, 0, 0)),
        ],
        out_specs=pl.BlockSpec((_BM, H), lambda i: (i, 0)),
        out_shape=jax.ShapeDtypeStruct((NP, H), jnp.float32),
    )(h_lin, deg_src)

    agg1 = _agg_call(h1s, edges)

    h2s = pl.pallas_call(
        _layer2_body,
        grid=(_G,),
        in_specs=[
            pl.BlockSpec((2, _BM, H), lambda i: (0, i, 0)),
            pl.BlockSpec((1, 1, _BM), lambda i: (i, 0, 0)),
            pl.BlockSpec((1, 1, _BM), lambda i: (i, 0, 0)),
            pl.BlockSpec((1, H), lambda i: (0, 0)),
            pl.BlockSpec((H, H), lambda i: (0, 0)),
        ],
        out_specs=pl.BlockSpec((_BM, H), lambda i: (i, 0)),
        out_shape=jax.ShapeDtypeStruct((NP, H), jnp.float32),
    )(agg1, deg_dst, deg_src, b1.reshape(1, H), W2)

    agg2 = _agg_call(h2s, edges)

    z_pad = pl.pallas_call(
        _z_body,
        grid=(_G,),
        in_specs=[
            pl.BlockSpec((2, _BM, H), lambda i: (0, i, 0)),
            pl.BlockSpec((1, 1, _BM), lambda i: (i, 0, 0)),
            pl.BlockSpec((1, H), lambda i: (0, 0)),
        ],
        out_specs=pl.BlockSpec((_BM, H), lambda i: (i, 0)),
        out_shape=jax.ShapeDtypeStruct((NP, H), jnp.float32),
    )(agg2, deg_dst, b2.reshape(1, H))

    adj = pl.pallas_call(
        _gram_body,
        grid=(N // _DBM, pl.cdiv(N, _DBN)),
        in_specs=[
            pl.BlockSpec((_DBM, H), lambda i, j: (i, 0)),
            pl.BlockSpec((_DBN, H), lambda i, j: (j, 0)),
        ],
        out_specs=pl.BlockSpec((_DBM, _DBN), lambda i, j: (i, j)),
        out_shape=jax.ShapeDtypeStruct((N, N), jnp.float32),
    )(z_pad, z_pad)
    return adj


# speedup vs baseline: 1.0271x; 1.0093x over previous
import functools

import jax
import jax.numpy as jnp
from jax import lax
from jax.experimental import pallas as pl
from jax.experimental.pallas import tpu as pltpu
from jax.experimental.pallas import tpu_sc as plsc

N = 10000
E = 320000
H = 128

NC = 2
NS = 16
LANES = 16

NP = 10240
TRASH = 10200

C = 128
CHUNKS_PER_TILE = 80
IB = 40
NROWS = NC * NS * CHUNKS_PER_TILE
E_PAD = NROWS * C
ROWS_PER_TILE_DEG = NROWS // NS

_mesh = plsc.VectorSubcoreMesh(core_axis_name="c", subcore_axis_name="s")


def _zero_vmem_rows(ref, nrows):
    def body(i, _):
        for k in range(H // LANES):
            ref[i, pl.ds(k * LANES, LANES)] = jnp.zeros((LANES,), jnp.float32)
        return _
    lax.fori_loop(0, nrows, body, None)


_DEG_K = 16

def _deg_body(edges_hbm, out_hbm, hist_sh, ones_v, zeros_v, idx_a, cp_v, sem):
    cid = lax.axis_index("c")
    sid = lax.axis_index("s")

    for k in range(C // LANES):
        ones_v[pl.ds(k * LANES, LANES)] = jnp.ones((LANES,), jnp.float32)
        zeros_v[pl.ds(k * LANES, LANES)] = jnp.zeros((LANES,), jnp.float32)

    slab = NP // NS
    base = sid * slab
    for k in range(slab // C):
        pltpu.sync_copy(zeros_v, hist_sh.at[pl.ds(base + k * C, C)])
    plsc.subcore_barrier()

    rbase = sid * ROWS_PER_TILE_DEG
    pltpu.sync_copy(edges_hbm.at[cid, pl.ds(rbase, ROWS_PER_TILE_DEG)], idx_a)
    def blk(b, carry):
        rb = b * _DEG_K
        for k in range(_DEG_K):
            pltpu.async_copy(ones_v, hist_sh.at[idx_a.at[rb + k]], sem, add=True)
        for k in range(_DEG_K):
            pltpu.make_async_copy(ones_v, hist_sh.at[idx_a.at[rb + k]], sem).wait()
        return carry
    lax.fori_loop(0, ROWS_PER_TILE_DEG // _DEG_K, blk, None)
    plsc.subcore_barrier()

    pltpu.sync_copy(hist_sh.at[pl.ds(base, slab)], cp_v)
    pltpu.sync_copy(cp_v, out_hbm.at[cid, pl.ds(base, slab)])


_deg_call = pl.kernel(
    _deg_body,
    out_type=jax.ShapeDtypeStruct((2, NP), jnp.float32),
    mesh=_mesh,
    scratch_types=[
        pltpu.VMEM_SHARED((NP,), jnp.float32),
        pltpu.VMEM((C,), jnp.float32),
        pltpu.VMEM((C,), jnp.float32),
        pltpu.VMEM((ROWS_PER_TILE_DEG, C), jnp.int32),
        pltpu.VMEM((NP // NS,), jnp.float32),
        pltpu.SemaphoreType.DMA,
    ],
)


def _agg_body(h_hbm, edges_hbm, out_hbm, acc_sh, rows0, rows1,
              sidx_a, didx_a, sem0, sem1):
    cid = lax.axis_index("c")
    sid = lax.axis_index("s")
    wid = cid * NS + sid

    _zero_vmem_rows(rows0, C)
    slab = NP // NS
    base = sid * slab
    for k in range(slab // C):
        pltpu.sync_copy(rows0, acc_sh.at[pl.ds(base + k * C, C)])
    plsc.subcore_barrier()

    wbase = wid * CHUNKS_PER_TILE
    def block(b, carry):
        bbase = wbase + b * IB
        pltpu.sync_copy(edges_hbm.at[0, pl.ds(bbase, IB)], sidx_a)
        pltpu.sync_copy(edges_hbm.at[1, pl.ds(bbase, IB)], didx_a)
        pltpu.async_copy(h_hbm.at[sidx_a.at[0]], rows0, sem0)
        def pair(i, c2):
            j0 = 2 * i
            pltpu.async_copy(h_hbm.at[sidx_a.at[j0 + 1]], rows1, sem1)
            pltpu.make_async_copy(h_hbm.at[sidx_a.at[j0]], rows0, sem0).wait()
            pltpu.sync_copy(rows0, acc_sh.at[didx_a.at[j0]], add=True)
            @pl.when(j0 + 2 < IB)
            def _():
                pltpu.async_copy(h_hbm.at[sidx_a.at[j0 + 2]], rows0, sem0)
            pltpu.make_async_copy(h_hbm.at[sidx_a.at[j0 + 1]], rows1, sem1).wait()
            pltpu.sync_copy(rows1, acc_sh.at[didx_a.at[j0 + 1]], add=True)
            return c2
        lax.fori_loop(0, IB // 2, pair, None)
        return carry
    lax.fori_loop(0, CHUNKS_PER_TILE // IB, block, None)
    plsc.subcore_barrier()

    for k in range(slab // C):
        pltpu.sync_copy(acc_sh.at[pl.ds(base + k * C, C)], rows0)
        pltpu.sync_copy(rows0, out_hbm.at[cid, pl.ds(base + k * C, C)])


_agg_call = pl.kernel(
    _agg_body,
    out_type=jax.ShapeDtypeStruct((2, NP, H), jnp.float32),
    mesh=_mesh,
    scratch_types=[
        pltpu.VMEM_SHARED((NP, H), jnp.float32),
        pltpu.VMEM((C, H), jnp.float32),
        pltpu.VMEM((C, H), jnp.float32),
        pltpu.VMEM((IB, C), jnp.int32),
        pltpu.VMEM((IB, C), jnp.int32),
        pltpu.SemaphoreType.DMA,
        pltpu.SemaphoreType.DMA,
    ],
)


_BM = 1280
_G = NP // _BM

def _h1_body(x_ref, w_ref, deg_ref, o_ref):
    ns = lax.rsqrt(jnp.maximum(deg_ref[0, 0, :], 1.0))
    h = jnp.dot(x_ref[...], w_ref[...], preferred_element_type=jnp.float32,
                precision=lax.Precision.HIGHEST)
    o_ref[...] = h * ns[:, None]


def _layer2_body(p_ref, degd_ref, degs_ref, b_ref, w_ref, o_ref):
    s = p_ref[0] + p_ref[1]
    nd = lax.rsqrt(jnp.maximum(degd_ref[0, 0, :], 1.0))
    h1 = jnp.maximum(s * nd[:, None] + b_ref[...], 0.0)
    ns = lax.rsqrt(jnp.maximum(degs_ref[0, 0, :], 1.0))
    h2 = jnp.dot(h1, w_ref[...], preferred_element_type=jnp.float32,
                 precision=lax.Precision.HIGHEST)
    o_ref[...] = h2 * ns[:, None]


def _z_body(p_ref, degd_ref, b_ref, o_ref):
    s = p_ref[0] + p_ref[1]
    nd = lax.rsqrt(jnp.maximum(degd_ref[0, 0, :], 1.0))
    o_ref[...] = s * nd[:, None] + b_ref[...]


_DBM = 2000
_DBN = 2560

def _gram_body(a_ref, b_ref, o_ref):
    o_ref[...] = lax.dot_general(
        a_ref[...], b_ref[...], (((1,), (1,)), ((), ())),
        preferred_element_type=jnp.float32, precision=lax.Precision.DEFAULT)


def kernel(in_feat, edge_index, W1, b1, W2, b2):
    pad = N + jnp.arange(E_PAD - E, dtype=jnp.int32) % (NP - N)
    src_p = jnp.concatenate([edge_index[0], pad]).reshape(NROWS, C)
    dst_p = jnp.concatenate([edge_index[1], pad]).reshape(NROWS, C)
    edges = jnp.stack([src_p, dst_p])

    deg = _deg_call(edges)
    deg_src = deg[0].reshape(_G, 1, _BM)
    deg_dst = deg[1].reshape(_G, 1, _BM)

    h1s = pl.pallas_call(
        _h1_body,
        grid=(_G,),
        in_specs=[
            pl.BlockSpec((_BM, H), lambda i: (i, 0)),
            pl.BlockSpec((H, H), lambda i: (0, 0)),
            pl.BlockSpec((1, 1, _BM), lambda i: (i, 0, 0)),
        ],
        out_specs=pl.BlockSpec((_BM, H), lambda i: (i, 0)),
        out_shape=jax.ShapeDtypeStruct((NP, H), jnp.float32),
    )(in_feat, W1, deg_src)

    agg1 = _agg_call(h1s, edges)

    h2s = pl.pallas_call(
        _layer2_body,
        grid=(_G,),
        in_specs=[
            pl.BlockSpec((2, _BM, H), lambda i: (0, i, 0)),
            pl.BlockSpec((1, 1, _BM), lambda i: (i, 0, 0)),
            pl.BlockSpec((1, 1, _BM), lambda i: (i, 0, 0)),
            pl.BlockSpec((1, H), lambda i: (0, 0)),
            pl.BlockSpec((H, H), lambda i: (0, 0)),
        ],
        out_specs=pl.BlockSpec((_BM, H), lambda i: (i, 0)),
        out_shape=jax.ShapeDtypeStruct((NP, H), jnp.float32),
    )(agg1, deg_dst, deg_src, b1.reshape(1, H), W2)

    agg2 = _agg_call(h2s, edges)

    z_pad = pl.pallas_call(
        _z_body,
        grid=(_G,),
        in_specs=[
            pl.BlockSpec((2, _BM, H), lambda i: (0, i, 0)),
            pl.BlockSpec((1, 1, _BM), lambda i: (i, 0, 0)),
            pl.BlockSpec((1, H), lambda i: (0, 0)),
        ],
        out_specs=pl.BlockSpec((_BM, H), lambda i: (i, 0)),
        out_shape=jax.ShapeDtypeStruct((NP, H), jnp.float32),
    )(agg2, deg_dst, b2.reshape(1, H))

    adj = pl.pallas_call(
        _gram_body,
        grid=(N // _DBM, pl.cdiv(N, _DBN)),
        in_specs=[
            pl.BlockSpec((_DBM, H), lambda i, j: (i, 0)),
            pl.BlockSpec((_DBN, H), lambda i, j: (j, 0)),
        ],
        out_specs=pl.BlockSpec((_DBM, _DBN), lambda i, j: (i, j)),
        out_shape=jax.ShapeDtypeStruct((N, N), jnp.float32),
    )(z_pad, z_pad)
    return adj


# speedup vs baseline: 1.0353x; 1.0080x over previous
import functools

import jax
import jax.numpy as jnp
from jax import lax
from jax.experimental import pallas as pl
from jax.experimental.pallas import tpu as pltpu
from jax.experimental.pallas import tpu_sc as plsc

N = 10000
E = 320000
H = 128

NC = 2
NS = 16
LANES = 16

NP = 10240
TRASH = 10200

C = 128
CHUNKS_PER_TILE = 80
IB = 40
NROWS = NC * NS * CHUNKS_PER_TILE
E_PAD = NROWS * C
ROWS_PER_TILE_DEG = NROWS // NS

_mesh = plsc.VectorSubcoreMesh(core_axis_name="c", subcore_axis_name="s")


def _zero_vmem_rows(ref, nrows):
    def body(i, _):
        for k in range(H // LANES):
            ref[i, pl.ds(k * LANES, LANES)] = jnp.zeros((LANES,), jnp.float32)
        return _
    lax.fori_loop(0, nrows, body, None)


_DEG_K = 16

def _deg_body(edges_hbm, out_hbm, hist_sh, ones_v, zeros_v, idx_a, cp_v, sem):
    cid = lax.axis_index("c")
    sid = lax.axis_index("s")

    for k in range(C // LANES):
        ones_v[pl.ds(k * LANES, LANES)] = jnp.ones((LANES,), jnp.float32)
        zeros_v[pl.ds(k * LANES, LANES)] = jnp.zeros((LANES,), jnp.float32)

    slab = NP // NS
    base = sid * slab
    for k in range(slab // C):
        pltpu.sync_copy(zeros_v, hist_sh.at[pl.ds(base + k * C, C)])
    plsc.subcore_barrier()

    rbase = sid * ROWS_PER_TILE_DEG
    pltpu.sync_copy(edges_hbm.at[cid, pl.ds(rbase, ROWS_PER_TILE_DEG)], idx_a)
    def blk(b, carry):
        rb = b * _DEG_K
        for k in range(_DEG_K):
            pltpu.async_copy(ones_v, hist_sh.at[idx_a.at[rb + k]], sem, add=True)
        for k in range(_DEG_K):
            pltpu.make_async_copy(ones_v, hist_sh.at[idx_a.at[rb + k]], sem).wait()
        return carry
    lax.fori_loop(0, ROWS_PER_TILE_DEG // _DEG_K, blk, None)
    plsc.subcore_barrier()

    pltpu.sync_copy(hist_sh.at[pl.ds(base, slab)], cp_v)
    pltpu.sync_copy(cp_v, out_hbm.at[cid, pl.ds(base, slab)])


_deg_call = pl.kernel(
    _deg_body,
    out_type=jax.ShapeDtypeStruct((2, NP), jnp.float32),
    mesh=_mesh,
    scratch_types=[
        pltpu.VMEM_SHARED((NP,), jnp.float32),
        pltpu.VMEM((C,), jnp.float32),
        pltpu.VMEM((C,), jnp.float32),
        pltpu.VMEM((ROWS_PER_TILE_DEG, C), jnp.int32),
        pltpu.VMEM((NP // NS,), jnp.float32),
        pltpu.SemaphoreType.DMA,
    ],
)


def _agg_body(h_hbm, edges_hbm, out_hbm, acc_sh, rows0, rows1,
              sidx_a, didx_a, sem0, sem1):
    cid = lax.axis_index("c")
    sid = lax.axis_index("s")
    wid = cid * NS + sid

    _zero_vmem_rows(rows0, C)
    slab = NP // NS
    base = sid * slab
    for k in range(slab // C):
        pltpu.sync_copy(rows0, acc_sh.at[pl.ds(base + k * C, C)])
    plsc.subcore_barrier()

    wbase = wid * CHUNKS_PER_TILE
    def block(b, carry):
        bbase = wbase + b * IB
        pltpu.sync_copy(edges_hbm.at[0, pl.ds(bbase, IB)], sidx_a)
        pltpu.sync_copy(edges_hbm.at[1, pl.ds(bbase, IB)], didx_a)
        pltpu.async_copy(h_hbm.at[sidx_a.at[0]], rows0, sem0)
        def pair(i, c2):
            j0 = 2 * i
            pltpu.async_copy(h_hbm.at[sidx_a.at[j0 + 1]], rows1, sem1)
            pltpu.make_async_copy(h_hbm.at[sidx_a.at[j0]], rows0, sem0).wait()
            pltpu.sync_copy(rows0, acc_sh.at[didx_a.at[j0]], add=True)
            @pl.when(j0 + 2 < IB)
            def _():
                pltpu.async_copy(h_hbm.at[sidx_a.at[j0 + 2]], rows0, sem0)
            pltpu.make_async_copy(h_hbm.at[sidx_a.at[j0 + 1]], rows1, sem1).wait()
            pltpu.sync_copy(rows1, acc_sh.at[didx_a.at[j0 + 1]], add=True)
            return c2
        lax.fori_loop(0, IB // 2, pair, None)
        return carry
    lax.fori_loop(0, CHUNKS_PER_TILE // IB, block, None)
    plsc.subcore_barrier()

    nk = slab // C
    for k in range(nk):
        r, sm = (rows0, sem0) if k % 2 == 0 else (rows1, sem1)
        if k >= 2:
            pltpu.make_async_copy(
                r, out_hbm.at[cid, pl.ds(base + (k - 2) * C, C)], sm).wait()
        pltpu.sync_copy(acc_sh.at[pl.ds(base + k * C, C)], r)
        pltpu.async_copy(r, out_hbm.at[cid, pl.ds(base + k * C, C)], sm)
    for k in range(nk - 2, nk):
        r, sm = (rows0, sem0) if k % 2 == 0 else (rows1, sem1)
        pltpu.make_async_copy(
            r, out_hbm.at[cid, pl.ds(base + k * C, C)], sm).wait()


_agg_call = pl.kernel(
    _agg_body,
    out_type=jax.ShapeDtypeStruct((2, NP, H), jnp.float32),
    mesh=_mesh,
    scratch_types=[
        pltpu.VMEM_SHARED((NP, H), jnp.float32),
        pltpu.VMEM((C, H), jnp.float32),
        pltpu.VMEM((C, H), jnp.float32),
        pltpu.VMEM((IB, C), jnp.int32),
        pltpu.VMEM((IB, C), jnp.int32),
        pltpu.SemaphoreType.DMA,
        pltpu.SemaphoreType.DMA,
    ],
)


_BM = 1280
_G = NP // _BM

def _h1_body(x_ref, w_ref, deg_ref, o_ref):
    ns = lax.rsqrt(jnp.maximum(deg_ref[0, 0, :], 1.0))
    h = jnp.dot(x_ref[...], w_ref[...], preferred_element_type=jnp.float32,
                precision=lax.Precision.HIGHEST)
    o_ref[...] = h * ns[:, None]


def _layer2_body(p_ref, degd_ref, degs_ref, b_ref, w_ref, o_ref):
    s = p_ref[0] + p_ref[1]
    nd = lax.rsqrt(jnp.maximum(degd_ref[0, 0, :], 1.0))
    h1 = jnp.maximum(s * nd[:, None] + b_ref[...], 0.0)
    ns = lax.rsqrt(jnp.maximum(degs_ref[0, 0, :], 1.0))
    h2 = jnp.dot(h1, w_ref[...], preferred_element_type=jnp.float32,
                 precision=lax.Precision.HIGHEST)
    o_ref[...] = h2 * ns[:, None]


def _z_body(p_ref, degd_ref, b_ref, o_ref):
    s = p_ref[0] + p_ref[1]
    nd = lax.rsqrt(jnp.maximum(degd_ref[0, 0, :], 1.0))
    o_ref[...] = s * nd[:, None] + b_ref[...]


_DBM = 2000
_DBN = 2560

def _gram_body(a_ref, b_ref, o_ref):
    o_ref[...] = lax.dot_general(
        a_ref[...], b_ref[...], (((1,), (1,)), ((), ())),
        preferred_element_type=jnp.float32, precision=lax.Precision.DEFAULT)


def kernel(in_feat, edge_index, W1, b1, W2, b2):
    pad = N + jnp.arange(E_PAD - E, dtype=jnp.int32) % (NP - N)
    src_p = jnp.concatenate([edge_index[0], pad]).reshape(NROWS, C)
    dst_p = jnp.concatenate([edge_index[1], pad]).reshape(NROWS, C)
    edges = jnp.stack([src_p, dst_p])

    deg = _deg_call(edges)
    deg_src = deg[0].reshape(_G, 1, _BM)
    deg_dst = deg[1].reshape(_G, 1, _BM)

    h1s = pl.pallas_call(
        _h1_body,
        grid=(_G,),
        in_specs=[
            pl.BlockSpec((_BM, H), lambda i: (i, 0)),
            pl.BlockSpec((H, H), lambda i: (0, 0)),
            pl.BlockSpec((1, 1, _BM), lambda i: (i, 0, 0)),
        ],
        out_specs=pl.BlockSpec((_BM, H), lambda i: (i, 0)),
        out_shape=jax.ShapeDtypeStruct((NP, H), jnp.float32),
    )(in_feat, W1, deg_src)

    agg1 = _agg_call(h1s, edges)

    h2s = pl.pallas_call(
        _layer2_body,
        grid=(_G,),
        in_specs=[
            pl.BlockSpec((2, _BM, H), lambda i: (0, i, 0)),
            pl.BlockSpec((1, 1, _BM), lambda i: (i, 0, 0)),
            pl.BlockSpec((1, 1, _BM), lambda i: (i, 0, 0)),
            pl.BlockSpec((1, H), lambda i: (0, 0)),
            pl.BlockSpec((H, H), lambda i: (0, 0)),
        ],
        out_specs=pl.BlockSpec((_BM, H), lambda i: (i, 0)),
        out_shape=jax.ShapeDtypeStruct((NP, H), jnp.float32),
    )(agg1, deg_dst, deg_src, b1.reshape(1, H), W2)

    agg2 = _agg_call(h2s, edges)

    z_pad = pl.pallas_call(
        _z_body,
        grid=(_G,),
        in_specs=[
            pl.BlockSpec((2, _BM, H), lambda i: (0, i, 0)),
            pl.BlockSpec((1, 1, _BM), lambda i: (i, 0, 0)),
            pl.BlockSpec((1, H), lambda i: (0, 0)),
        ],
        out_specs=pl.BlockSpec((_BM, H), lambda i: (i, 0)),
        out_shape=jax.ShapeDtypeStruct((NP, H), jnp.float32),
    )(agg2, deg_dst, b2.reshape(1, H))

    adj = pl.pallas_call(
        _gram_body,
        grid=(N // _DBM, pl.cdiv(N, _DBN)),
        in_specs=[
            pl.BlockSpec((_DBM, H), lambda i, j: (i, 0)),
            pl.BlockSpec((_DBN, H), lambda i, j: (j, 0)),
        ],
        out_specs=pl.BlockSpec((_DBM, _DBN), lambda i, j: (i, j)),
        out_shape=jax.ShapeDtypeStruct((N, N), jnp.float32),
    )(z_pad, z_pad)
    return adj
